# baseline (speedup 1.0000x reference)
import jax
import jax.numpy as jnp
from jax import lax
from jax.experimental import pallas as pl
from jax.experimental.pallas import tpu as pltpu
from jax.experimental.pallas import tpu_sc as plsc

BATCH = 16384
DIM = 32
NROWS = 1000000
PACK = 128 // DIM
NBLK = 245
NPACKED = NBLK * 1024
NC, NS = 2, 16
NW = NC * NS
ROWS_PER_W = BATCH // NW
RB = 4096


def _repack_body(u_ref, i_ref, eye_ref, uo_ref, io_ref):
    dn = (((0,), (0,)), ((), ()))
    q = RB // PACK
    for src, dst in ((u_ref, uo_ref), (i_ref, io_ref)):
        t = lax.dot_general(src[...], eye_ref[...], dn,
                            preferred_element_type=jnp.float32)
        dst[...] = jnp.concatenate([t[a * q:(a + 1) * q] for a in range(PACK)],
                                   axis=1)


def _tc_repack(utabT, itabT):
    grid = (NBLK,)
    in_spec = pl.BlockSpec((DIM, RB), lambda b: (0, b))
    eye_spec = pl.BlockSpec((DIM, DIM), lambda b: (0, 0))
    out_spec = pl.BlockSpec((RB // PACK, 128), lambda b: (b, 0))
    return pl.pallas_call(
        _repack_body,
        grid=grid,
        in_specs=[in_spec, in_spec, eye_spec],
        out_specs=(out_spec, out_spec),
        out_shape=(
            jax.ShapeDtypeStruct((NPACKED, 128), jnp.float32),
            jax.ShapeDtypeStruct((NPACKED, 128), jnp.float32),
        ),
    )(utabT, itabT, jnp.eye(DIM, dtype=jnp.float32))


def _gather_tower(ids_hbm, packed, outT, idx_s, cols_v, slab_v, sem, base):
    pltpu.sync_copy(ids_hbm.at[pl.ds(base, ROWS_PER_W)], idx_s)
    kvec = lax.iota(jnp.int32, 16)

    def fire(i, _):
        vec = idx_s[pl.ds(i * 16, 16)]
        qvec = (lax.shift_right_logical(vec, 12) * 1024
                + lax.bitwise_and(vec, 1023))
        pltpu.async_copy(packed.at[qvec], slab_v.at[pl.ds(i * 16, 16)], sem)
        return _

    lax.fori_loop(0, ROWS_PER_W // 16, fire, None)
    pltpu.make_async_copy(packed.at[pl.ds(0, ROWS_PER_W)], slab_v, sem).wait()

    def extract(i, _):
        b = i * 16
        vec = idx_s[pl.ds(b, 16)]
        cvec = lax.bitwise_and(lax.shift_right_logical(vec, 10), 3) * DIM
        for j in range(16):
            jv = jnp.full((16,), b + j, jnp.int32)
            cb = cvec[j] + kvec
            rv = jnp.full((16,), b + j, jnp.int32)
            lo = plsc.load_gather(slab_v, [jv, cb])
            hi = plsc.load_gather(slab_v, [jv, cb + 16])
            plsc.store_scatter(cols_v, [kvec, rv], lo)
            plsc.store_scatter(cols_v, [kvec + 16, rv], hi)
        return _

    lax.fori_loop(0, ROWS_PER_W // 16, extract, None)
    pltpu.sync_copy(cols_v, outT.at[:, pl.ds(base, ROWS_PER_W)])


def _sc_gather_body(uids, iids, upacked, ipacked, u_outT, i_outT,
                    uidx_s, iidx_s, ucols_v, icols_v, slab_v, sem):
    w = lax.axis_index("s") * NC + lax.axis_index("c")
    base = w * ROWS_PER_W
    _gather_tower(uids, upacked, u_outT, uidx_s, ucols_v, slab_v, sem, base)
    _gather_tower(iids, ipacked, i_outT, iidx_s, icols_v, slab_v, sem, base)


def _sc_gather(uids, iids, upacked, ipacked):
    mesh = plsc.VectorSubcoreMesh(core_axis_name="c", subcore_axis_name="s")
    f = pl.kernel(
        _sc_gather_body,
        out_type=(
            jax.ShapeDtypeStruct((DIM, BATCH), jnp.float32),
            jax.ShapeDtypeStruct((DIM, BATCH), jnp.float32),
        ),
        mesh=mesh,
        scratch_types=[
            pltpu.VMEM((ROWS_PER_W,), jnp.int32),
            pltpu.VMEM((ROWS_PER_W,), jnp.int32),
            pltpu.VMEM((DIM, ROWS_PER_W), jnp.float32),
            pltpu.VMEM((DIM, ROWS_PER_W), jnp.float32),
            pltpu.VMEM((ROWS_PER_W, 128), jnp.float32),
            pltpu.SemaphoreType.DMA,
        ],
        compiler_params=pltpu.CompilerParams(needs_layout_passes=False),
    )
    return f(uids, iids, upacked, ipacked)


def _tc_proj_body(u_ref, i_ref, uW_ref, ub_ref, iW_ref, ib_ref, uo_ref, io_ref):
    dn = (((1,), (0,)), ((), ()))
    uo_ref[...] = lax.dot_general(uW_ref[...], u_ref[...], dn,
                                  preferred_element_type=jnp.float32) + ub_ref[...]
    io_ref[...] = lax.dot_general(iW_ref[...], i_ref[...], dn,
                                  preferred_element_type=jnp.float32) + ib_ref[...]


def _tc_proj(uT, iT, user_W, user_b, item_W, item_b):
    blk = 2048
    grid = (BATCH // blk,)
    col_spec = pl.BlockSpec((DIM, blk), lambda b: (0, b))
    w_spec = pl.BlockSpec((DIM, DIM), lambda b: (0, 0))
    b_spec = pl.BlockSpec((DIM, 1), lambda b: (0, 0))
    return pl.pallas_call(
        _tc_proj_body,
        grid=grid,
        in_specs=[col_spec, col_spec, w_spec, b_spec, w_spec, b_spec],
        out_specs=(col_spec, col_spec),
        out_shape=(
            jax.ShapeDtypeStruct((DIM, BATCH), jnp.float32),
            jax.ShapeDtypeStruct((DIM, BATCH), jnp.float32),
        ),
    )(uT, iT, user_W, user_b.reshape(DIM, 1), item_W, item_b.reshape(DIM, 1))


def kernel(user_ids, item_ids, user_table, item_table, user_W, user_b, item_W, item_b):
    upacked, ipacked = _tc_repack(user_table.T, item_table.T)
    uT, iT = _sc_gather(user_ids.astype(jnp.int32),
                        item_ids.astype(jnp.int32),
                        upacked, ipacked)
    u_projT, i_projT = _tc_proj(uT, iT, user_W, user_b, item_W, item_b)
    return (u_projT.T, i_projT.T)


# speedup vs baseline: 1.9714x; 1.9714x over previous
import jax
import jax.numpy as jnp
from jax import lax
from jax.experimental import pallas as pl
from jax.experimental.pallas import tpu as pltpu
from jax.experimental.pallas import tpu_sc as plsc

BATCH = 16384
DIM = 32
NC, NS = 2, 16
NW = NC * NS
ROWS_PER_W = BATCH // NW
RING = 16


def _gather_tower(ids_hbm, tabT, outT, idx_s, cols_v, slab_v, sem, base):
    pltpu.sync_copy(ids_hbm.at[pl.ds(base, ROWS_PER_W)], idx_s)
    kvec = lax.iota(jnp.int32, 16)

    def round_(i, _):
        b = i * RING
        vec = idx_s[pl.ds(b, RING)]
        qvec = lax.shift_right_logical(vec, 7)
        cvec = lax.bitwise_and(vec, 127)
        for j in range(RING):
            off = pl.multiple_of(qvec[j] * 128, 128)
            pltpu.async_copy(tabT.at[:, pl.ds(off, 128)], slab_v.at[j], sem)
        for j in range(RING):
            pltpu.make_async_copy(tabT.at[:, pl.ds(0, 128)], slab_v.at[j],
                                  sem).wait()
        for j in range(RING):
            jv = jnp.full((16,), j, jnp.int32)
            cv = jnp.full((16,), cvec[j], jnp.int32)
            rv = jnp.full((16,), b + j, jnp.int32)
            lo = plsc.load_gather(slab_v, [jv, kvec, cv])
            hi = plsc.load_gather(slab_v, [jv, kvec + 16, cv])
            plsc.store_scatter(cols_v, [kvec, rv], lo)
            plsc.store_scatter(cols_v, [kvec + 16, rv], hi)
        return _

    lax.fori_loop(0, ROWS_PER_W // RING, round_, None)
    pltpu.sync_copy(cols_v, outT.at[:, pl.ds(base, ROWS_PER_W)])


def _sc_gather_body(uids, iids, utabT, itabT, u_outT, i_outT,
                    uidx_s, iidx_s, ucols_v, icols_v, slab_v, sem):
    w = lax.axis_index("s") * NC + lax.axis_index("c")
    base = w * ROWS_PER_W
    _gather_tower(uids, utabT, u_outT, uidx_s, ucols_v, slab_v, sem, base)
    _gather_tower(iids, itabT, i_outT, iidx_s, icols_v, slab_v, sem, base)


def _sc_gather(uids, iids, utabT, itabT):
    mesh = plsc.VectorSubcoreMesh(core_axis_name="c", subcore_axis_name="s")
    f = pl.kernel(
        _sc_gather_body,
        out_type=(
            jax.ShapeDtypeStruct((DIM, BATCH), jnp.float32),
            jax.ShapeDtypeStruct((DIM, BATCH), jnp.float32),
        ),
        mesh=mesh,
        scratch_types=[
            pltpu.VMEM((ROWS_PER_W,), jnp.int32),
            pltpu.VMEM((ROWS_PER_W,), jnp.int32),
            pltpu.VMEM((DIM, ROWS_PER_W), jnp.float32),
            pltpu.VMEM((DIM, ROWS_PER_W), jnp.float32),
            pltpu.VMEM((RING, DIM, 128), jnp.float32),
            pltpu.SemaphoreType.DMA,
        ],
        compiler_params=pltpu.CompilerParams(needs_layout_passes=False),
    )
    return f(uids, iids, utabT, itabT)


def _tc_proj_body(u_ref, i_ref, uW_ref, ub_ref, iW_ref, ib_ref, uo_ref, io_ref):
    dn = (((1,), (0,)), ((), ()))
    uo_ref[...] = lax.dot_general(uW_ref[...], u_ref[...], dn,
                                  preferred_element_type=jnp.float32) + ub_ref[...]
    io_ref[...] = lax.dot_general(iW_ref[...], i_ref[...], dn,
                                  preferred_element_type=jnp.float32) + ib_ref[...]


def _tc_proj(uT, iT, user_W, user_b, item_W, item_b):
    blk = 2048
    grid = (BATCH // blk,)
    col_spec = pl.BlockSpec((DIM, blk), lambda b: (0, b))
    w_spec = pl.BlockSpec((DIM, DIM), lambda b: (0, 0))
    b_spec = pl.BlockSpec((DIM, 1), lambda b: (0, 0))
    return pl.pallas_call(
        _tc_proj_body,
        grid=grid,
        in_specs=[col_spec, col_spec, w_spec, b_spec, w_spec, b_spec],
        out_specs=(col_spec, col_spec),
        out_shape=(
            jax.ShapeDtypeStruct((DIM, BATCH), jnp.float32),
            jax.ShapeDtypeStruct((DIM, BATCH), jnp.float32),
        ),
    )(uT, iT, user_W, user_b.reshape(DIM, 1), item_W, item_b.reshape(DIM, 1))


def kernel(user_ids, item_ids, user_table, item_table, user_W, user_b, item_W, item_b):
    uT, iT = _sc_gather(user_ids.astype(jnp.int32),
                        item_ids.astype(jnp.int32),
                        user_table.T, item_table.T)
    u_projT, i_projT = _tc_proj(uT, iT, user_W, user_b, item_W, item_b)
    return (u_projT.T, i_projT.T)


# speedup vs baseline: 2.0593x; 1.0446x over previous
import jax
import jax.numpy as jnp
from jax import lax
from jax.experimental import pallas as pl
from jax.experimental.pallas import tpu as pltpu
from jax.experimental.pallas import tpu_sc as plsc

BATCH = 16384
DIM = 32
NC, NS = 2, 16
NW = NC * NS
ROWS_PER_W = BATCH // NW
RND = ROWS_PER_W // 16


def _gather_tower(ids_hbm, tabT, outT, idx_s, cols_v, slab_v, semA, semB, base):
    pltpu.sync_copy(ids_hbm.at[pl.ds(base, ROWS_PER_W)], idx_s)
    kvec = lax.iota(jnp.int32, 16)

    def fire(vec, half, sem):
        qvec = lax.shift_right_logical(vec, 7)
        for j in range(8):
            slot = half * 8 + j
            off = pl.multiple_of(qvec[slot] * 128, 128)
            pltpu.async_copy(tabT.at[:, pl.ds(off, 128)],
                             slab_v.at[:, pl.ds(slot * 128, 128)],
                             sem)

    def drain(half, sem):
        pltpu.make_async_copy(tabT.at[:, pl.ds(0, 1024)],
                              slab_v.at[:, pl.ds(half * 1024, 1024)],
                              sem).wait()

    def extract(vec, half, b):
        cvec = lax.bitwise_and(vec, 127)
        for j in range(8):
            slot = half * 8 + j
            cb = slot * 128 + cvec[slot]
            cv = jnp.full((16,), cb, jnp.int32)
            rv = jnp.full((16,), b + slot, jnp.int32)
            lo = plsc.load_gather(slab_v, [kvec, cv])
            hi = plsc.load_gather(slab_v, [kvec + 16, cv])
            plsc.store_scatter(cols_v, [kvec, rv], lo)
            plsc.store_scatter(cols_v, [kvec + 16, rv], hi)

    vec0 = idx_s[pl.ds(0, 16)]
    fire(vec0, 0, semA)
    fire(vec0, 1, semB)

    def round_(r, vec):
        nvec = idx_s[pl.ds((r + 1) * 16, 16)]
        drain(0, semA)
        extract(vec, 0, r * 16)
        fire(nvec, 0, semA)
        drain(1, semB)
        extract(vec, 1, r * 16)
        fire(nvec, 1, semB)
        return nvec

    last = lax.fori_loop(0, RND - 1, round_, vec0)
    drain(0, semA)
    extract(last, 0, (RND - 1) * 16)
    drain(1, semB)
    extract(last, 1, (RND - 1) * 16)
    pltpu.sync_copy(cols_v, outT.at[:, pl.ds(base, ROWS_PER_W)])


def _sc_gather_body(uids, iids, utabT, itabT, u_outT, i_outT,
                    uidx_s, iidx_s, cols_v, slab_v, semA, semB):
    w = lax.axis_index("s") * NC + lax.axis_index("c")
    base = w * ROWS_PER_W
    _gather_tower(uids, utabT, u_outT, uidx_s, cols_v, slab_v, semA, semB, base)
    _gather_tower(iids, itabT, i_outT, iidx_s, cols_v, slab_v, semA, semB, base)


def _sc_gather(uids, iids, utabT, itabT):
    mesh = plsc.VectorSubcoreMesh(core_axis_name="c", subcore_axis_name="s")
    f = pl.kernel(
        _sc_gather_body,
        out_type=(
            jax.ShapeDtypeStruct((DIM, BATCH), jnp.float32),
            jax.ShapeDtypeStruct((DIM, BATCH), jnp.float32),
        ),
        mesh=mesh,
        scratch_types=[
            pltpu.VMEM((ROWS_PER_W,), jnp.int32),
            pltpu.VMEM((ROWS_PER_W,), jnp.int32),
            pltpu.VMEM((DIM, ROWS_PER_W), jnp.float32),
            pltpu.VMEM((DIM, 2048), jnp.float32),
            pltpu.SemaphoreType.DMA,
            pltpu.SemaphoreType.DMA,
        ],
        compiler_params=pltpu.CompilerParams(needs_layout_passes=False),
    )
    return f(uids, iids, utabT, itabT)


def _tc_proj_body(u_ref, i_ref, uW_ref, ub_ref, iW_ref, ib_ref, uo_ref, io_ref):
    dn = (((1,), (0,)), ((), ()))
    uo_ref[...] = lax.dot_general(uW_ref[...], u_ref[...], dn,
                                  preferred_element_type=jnp.float32) + ub_ref[...]
    io_ref[...] = lax.dot_general(iW_ref[...], i_ref[...], dn,
                                  preferred_element_type=jnp.float32) + ib_ref[...]


def _tc_proj(uT, iT, user_W, user_b, item_W, item_b):
    blk = 2048
    grid = (BATCH // blk,)
    col_spec = pl.BlockSpec((DIM, blk), lambda b: (0, b))
    w_spec = pl.BlockSpec((DIM, DIM), lambda b: (0, 0))
    b_spec = pl.BlockSpec((DIM, 1), lambda b: (0, 0))
    return pl.pallas_call(
        _tc_proj_body,
        grid=grid,
        in_specs=[col_spec, col_spec, w_spec, b_spec, w_spec, b_spec],
        out_specs=(col_spec, col_spec),
        out_shape=(
            jax.ShapeDtypeStruct((DIM, BATCH), jnp.float32),
            jax.ShapeDtypeStruct((DIM, BATCH), jnp.float32),
        ),
    )(uT, iT, user_W, user_b.reshape(DIM, 1), item_W, item_b.reshape(DIM, 1))


def kernel(user_ids, item_ids, user_table, item_table, user_W, user_b, item_W, item_b):
    uT, iT = _sc_gather(user_ids.astype(jnp.int32),
                        item_ids.astype(jnp.int32),
                        user_table.T, item_table.T)
    u_projT, i_projT = _tc_proj(uT, iT, user_W, user_b, item_W, item_b)
    return (u_projT.T, i_projT.T)
